# baseline (speedup 1.0000x reference)
import functools

import jax
import jax.numpy as jnp
from jax import lax
from jax.experimental import pallas as pl
from jax.experimental.pallas import tpu as pltpu
from jax.experimental.pallas import tpu_sc as plsc

NC, NS, LANES = 2, 16, 16
NW = NC * NS
CH = 128
TILE_E = 3200
TILE_N = 2000


def _xsd_body(x_ref, wb_ref, wc_ref, xs_ref, xd_ref):
    xv = x_ref[...]
    xs_ref[...] = jnp.dot(xv, wb_ref[...], preferred_element_type=jnp.float32)
    xd_ref[...] = jnp.dot(xv, wc_ref[...], preferred_element_type=jnp.float32)


def _edge_body(ea_ref, g_ref, w1_ref, b1_ref, w2_ref, b2_ref, out_ref):
    w1 = w1_ref[...].astype(jnp.bfloat16)
    w2 = w2_ref[...].astype(jnp.bfloat16)
    t = (jnp.dot(ea_ref[...].astype(jnp.bfloat16), w1,
                 preferred_element_type=jnp.float32)
         + g_ref[...] + b1_ref[...])
    h = t * jax.nn.sigmoid(t)
    out_ref[...] = (jnp.dot(h.astype(jnp.bfloat16), w2,
                            preferred_element_type=jnp.float32)
                    + b2_ref[...])


def _node_body(x_ref, agg_ref, aggb_ref, w1a_ref, w1b_ref, b1_ref, w2_ref,
               b2_ref, out_ref):
    agg = (agg_ref[0] + agg_ref[1]) + (aggb_ref[0] + aggb_ref[1])
    t = (jnp.dot(x_ref[...], w1a_ref[...], preferred_element_type=jnp.float32)
         + jnp.dot(agg, w1b_ref[...], preferred_element_type=jnp.float32)
         + b1_ref[...])
    h = t * jax.nn.sigmoid(t)
    out_ref[...] = (jnp.dot(h, w2_ref[...], preferred_element_type=jnp.float32)
                    + b2_ref[...])


def _run_xsd(x2, wb, wc):
    n, h = x2.shape
    wspec = pl.BlockSpec((h, h), lambda i: (0, 0))
    return pl.pallas_call(
        _xsd_body,
        grid=(n // TILE_N,),
        in_specs=[pl.BlockSpec((TILE_N, h), lambda i: (i, 0)), wspec, wspec],
        out_specs=[pl.BlockSpec((TILE_N, h), lambda i: (i, 0))] * 2,
        out_shape=[jax.ShapeDtypeStruct((n, h), jnp.float32)] * 2,
    )(x2, wb, wc)


def _run_edge_half(ea_full, ea_off, g, w1a, b1, w2, b2):
    eh, h = g.shape
    espec = pl.BlockSpec((TILE_E, h), lambda i: (i + ea_off, 0))
    gspec = pl.BlockSpec((TILE_E, h), lambda i: (i, 0))
    wspec = pl.BlockSpec((h, h), lambda i: (0, 0))
    bspec = pl.BlockSpec((1, h), lambda i: (0, 0))
    return pl.pallas_call(
        _edge_body,
        grid=(eh // TILE_E,),
        in_specs=[espec, gspec, wspec, bspec, wspec, bspec],
        out_specs=gspec,
        out_shape=jax.ShapeDtypeStruct((eh, h), jnp.float32),
    )(ea_full, g, w1a, b1, w2, b2)


def _run_edge_into(buf, ea_in, g, w1a, b1, w2, b2, out_off, e_total, in_off):
    eh, h = g.shape
    espec = pl.BlockSpec((TILE_E, h), lambda i: (i + in_off, 0))
    gspec = pl.BlockSpec((TILE_E, h), lambda i: (i, 0))
    ospec = pl.BlockSpec((TILE_E, h), lambda i: (i + out_off, 0))
    wspec = pl.BlockSpec((h, h), lambda i: (0, 0))
    bspec = pl.BlockSpec((1, h), lambda i: (0, 0))
    ins = [ea_in, g, w1a, b1, w2, b2]
    specs = [espec, gspec, wspec, bspec, wspec, bspec]
    if buf is None:
        body = _edge_body
        aliases = {}
    else:
        def body(_, *refs):
            _edge_body(*refs)
        ins = [buf] + ins
        specs = [pl.BlockSpec(memory_space=pltpu.MemorySpace.HBM)] + specs
        aliases = {0: 0}
    return pl.pallas_call(
        body,
        grid=(eh // TILE_E,),
        in_specs=specs,
        out_specs=ospec,
        out_shape=jax.ShapeDtypeStruct((e_total, h), jnp.float32),
        input_output_aliases=aliases,
    )(*ins)


def _run_node(x2, agg2a, agg2b, w1a, w1b, b1, w2, b2):
    n, h = x2.shape
    nspec = pl.BlockSpec((TILE_N, h), lambda i: (i, 0))
    aspec = pl.BlockSpec((2, TILE_N, h), lambda i: (0, i, 0))
    wspec = pl.BlockSpec((h, h), lambda i: (0, 0))
    bspec = pl.BlockSpec((1, h), lambda i: (0, 0))
    return pl.pallas_call(
        _node_body,
        grid=(n // TILE_N,),
        in_specs=[nspec, aspec, aspec, wspec, wspec, bspec, wspec, bspec],
        out_specs=nspec,
        out_shape=jax.ShapeDtypeStruct((n, h), jnp.float32),
    )(x2, agg2a, agg2b, w1a, w1b, b1, w2, b2)


def _make_gather(n_pad, e, h, k0):
    epw = e // NW
    nch = epw // CH
    tail = epw - nch * CH
    npair = nch // 2
    npc = n_pad // NS
    mesh = plsc.VectorSubcoreMesh(core_axis_name="c", subcore_axis_name="s")

    @functools.partial(
        pl.kernel, mesh=mesh,
        out_type=jax.ShapeDtypeStruct((e, h), jnp.float32),
        scratch_types=[
            pltpu.VMEM((epw,), jnp.int32),
            pltpu.VMEM((epw,), jnp.int32),
            pltpu.VMEM((CH, h), jnp.float32),
            pltpu.VMEM((CH, h), jnp.float32),
            pltpu.VMEM((CH, h), jnp.float32),
            pltpu.VMEM((CH, h), jnp.float32),
            pltpu.VMEM((tail, h), jnp.float32),
            pltpu.VMEM((tail, h), jnp.float32),
            pltpu.SemaphoreType.DMA,
            pltpu.SemaphoreType.DMA,
            pltpu.SemaphoreType.DMA,
        ])
    def gather(xs_hbm, xd_hbm, src_hbm, dst_hbm, g_hbm,
               sidx, didx, rs0, rd0, rs1, rd1, ts, td,
               sem0, sem1, semt):
        c = lax.axis_index("c")
        s = lax.axis_index("s")
        wid = s * NC + c
        base = wid * epw
        pltpu.sync_copy(src_hbm.at[pl.ds(k0 + base, epw)], sidx)
        pltpu.sync_copy(dst_hbm.at[pl.ds(k0 + base, epw)], didx)

        def issue(chunk, rs, rd, sem):
            off = pl.multiple_of(chunk * CH, CH)
            pltpu.async_copy(xs_hbm.at[sidx.at[pl.ds(off, CH)]], rs, sem)
            pltpu.async_copy(xd_hbm.at[didx.at[pl.ds(off, CH)]], rd, sem)

        def consume(chunk, rs, rd, sem):
            off = pl.multiple_of(chunk * CH, CH)
            pltpu.make_async_copy(
                xs_hbm.at[sidx.at[pl.ds(off, CH)]], rs, sem).wait()
            pltpu.make_async_copy(
                xd_hbm.at[didx.at[pl.ds(off, CH)]], rd, sem).wait()

            def add_row(i, carry2):
                for k in range(h // LANES):
                    sl = pl.ds(k * LANES, LANES)
                    rs[i, sl] = rs[i, sl] + rd[i, sl]
                return carry2

            lax.fori_loop(0, CH, add_row, 0)
            pltpu.sync_copy(rs, g_hbm.at[pl.ds(base + off, CH)])

        toff = nch * CH
        pltpu.async_copy(xs_hbm.at[sidx.at[pl.ds(toff, tail)]], ts, semt)
        pltpu.async_copy(xd_hbm.at[didx.at[pl.ds(toff, tail)]], td, semt)
        issue(0, rs0, rd0, sem0)

        def pair(j, carry):
            issue(2 * j + 1, rs1, rd1, sem1)
            consume(2 * j, rs0, rd0, sem0)

            @pl.when(2 * j + 2 < nch)
            def _():
                issue(2 * j + 2, rs0, rd0, sem0)

            consume(2 * j + 1, rs1, rd1, sem1)
            return carry

        lax.fori_loop(0, npair, pair, 0)
        if nch % 2 == 1:
            consume(nch - 1, rs0, rd0, sem0)

        pltpu.make_async_copy(
            xs_hbm.at[sidx.at[pl.ds(toff, tail)]], ts, semt).wait()
        pltpu.make_async_copy(
            xd_hbm.at[didx.at[pl.ds(toff, tail)]], td, semt).wait()

        def add_tail(i, carry2):
            for k in range(h // LANES):
                sl = pl.ds(k * LANES, LANES)
                ts[i, sl] = ts[i, sl] + td[i, sl]
            return carry2

        lax.fori_loop(0, tail, add_tail, 0)
        pltpu.sync_copy(ts, g_hbm.at[pl.ds(base + toff, tail)])

    return gather


def _make_scatter(n_pad, e, h, k0, ne_off):
    epw = e // NW
    npc = n_pad // NS
    mesh = plsc.VectorSubcoreMesh(core_axis_name="c", subcore_axis_name="s")

    nch = epw // CH
    tail = epw - nch * CH
    npair = nch // 2

    @functools.partial(
        pl.kernel, mesh=mesh,
        out_type=jax.ShapeDtypeStruct((NC, n_pad, h), jnp.float32),
        scratch_types=[
            pltpu.VMEM((CH,), jnp.int32),
            pltpu.VMEM((CH,), jnp.int32),
            pltpu.VMEM((tail,), jnp.int32),
            pltpu.VMEM((CH, h), jnp.float32),
            pltpu.VMEM((CH, h), jnp.float32),
            pltpu.VMEM((tail, h), jnp.float32),
            pltpu.VMEM_SHARED((n_pad, h), jnp.float32),
            pltpu.SemaphoreType.DMA,
            pltpu.SemaphoreType.DMA,
            pltpu.SemaphoreType.DMA,
        ])
    def scatter(ne_hbm, dst_hbm, zeros_hbm, agg_hbm,
                idx0, idx1, idxt, rows0, rows1, rowst, acc,
                sem0, sem1, semt):
        c = lax.axis_index("c")
        s = lax.axis_index("s")
        wid = s * NC + c
        base = wid * epw

        def issue(chunk, idxs, rows, sem, nrow):
            b = pl.multiple_of(chunk * CH, 8)
            pltpu.async_copy(dst_hbm.at[pl.ds(k0 + base + b, nrow)], idxs, sem)
            pltpu.async_copy(ne_hbm.at[pl.ds(ne_off + base + b, nrow)],
                             rows, sem)

        def consume(chunk, idxs, rows, sem, nrow):
            b = pl.multiple_of(chunk * CH, 8)
            pltpu.make_async_copy(
                dst_hbm.at[pl.ds(k0 + base + b, nrow)], idxs, sem).wait()
            pltpu.make_async_copy(
                ne_hbm.at[pl.ds(ne_off + base + b, nrow)], rows, sem).wait()
            pltpu.sync_copy(rows, acc.at[idxs], add=True)

        issue(nch, idxt, rowst, semt, tail)
        issue(0, idx0, rows0, sem0, CH)

        pltpu.sync_copy(zeros_hbm.at[pl.ds(s * npc, npc)],
                        acc.at[pl.ds(s * npc, npc)])
        plsc.subcore_barrier()

        def pair(j, carry):
            issue(2 * j + 1, idx1, rows1, sem1, CH)
            consume(2 * j, idx0, rows0, sem0, CH)

            @pl.when(2 * j + 2 < nch)
            def _():
                issue(2 * j + 2, idx0, rows0, sem0, CH)

            consume(2 * j + 1, idx1, rows1, sem1, CH)
            return carry

        lax.fori_loop(0, npair, pair, 0)
        if nch % 2 == 1:
            consume(nch - 1, idx0, rows0, sem0, CH)
        consume(nch, idxt, rowst, semt, tail)
        plsc.subcore_barrier()
        pltpu.sync_copy(acc.at[pl.ds(s * npc, npc)],
                        agg_hbm.at[c, pl.ds(s * npc, npc)])

    return scatter


def kernel(x, edge_index, edge_attr, node_positions,
           edge_W1, edge_b1, edge_W2, edge_b2,
           node_W1, node_b1, node_W2, node_b2):
    del node_positions
    b, n, h = x.shape
    e = edge_index.shape[1]
    n_layers = edge_W1.shape[0]

    x2 = x[0]
    src = edge_index[0]
    dst = edge_index[1]
    n_pad = -(-n // (8 * NS)) * (8 * NS)
    zeros_n = jnp.zeros((n_pad, h), jnp.float32)

    K = 2
    eh = e // K
    nblk = eh // TILE_E
    gathers = [_make_gather(n_pad, eh, h, k * eh) for k in range(K)]
    scat_half = [_make_scatter(n_pad, eh, h, k * eh, 0) for k in range(K)]
    scat_full = [_make_scatter(n_pad, eh, h, k * eh, k * eh) for k in range(K)]
    ea_full = edge_attr[0]
    ea_halves = None

    for i in range(n_layers):
        w1a = edge_W1[i, :h]
        wb = edge_W1[i, h:2 * h]
        wc = edge_W1[i, 2 * h:]
        b1 = edge_b1[i][None]
        w2 = edge_W2[i]
        b2 = edge_b2[i][None]
        xs, xd = _run_xsd(x2, wb, wc)
        gs = [gathers[k](xs, xd, src, dst) for k in range(K)]
        if i < n_layers - 1:
            if ea_halves is None:
                ea_halves = [_run_edge_half(ea_full, k * nblk, gs[k], w1a, b1,
                                            w2, b2) for k in range(K)]
            else:
                ea_halves = [_run_edge_half(ea_halves[k], 0, gs[k], w1a, b1,
                                            w2, b2) for k in range(K)]
            aggs = [scat_half[k](ea_halves[k], dst, zeros_n)
                    for k in range(K)]
        else:
            buf = None
            for k in range(K):
                if ea_halves is not None:
                    ea_in, in_off = ea_halves[k], 0
                else:
                    ea_in, in_off = ea_full, k * nblk
                buf = _run_edge_into(buf, ea_in, gs[k], w1a, b1, w2, b2,
                                     k * nblk, e, in_off)
            ea_full = buf
            aggs = [scat_full[k](ea_full, dst, zeros_n) for k in range(K)]
        x2 = _run_node(x2, aggs[0], aggs[1], node_W1[i, :h], node_W1[i, h:],
                       node_b1[i][None], node_W2[i], node_b2[i][None])

    return (x2[None], ea_full[None])


# speedup vs baseline: 1.0241x; 1.0241x over previous
import functools

import jax
import jax.numpy as jnp
from jax import lax
from jax.experimental import pallas as pl
from jax.experimental.pallas import tpu as pltpu
from jax.experimental.pallas import tpu_sc as plsc

NC, NS, LANES = 2, 16, 16
NW = NC * NS
CH = 128
TILE_E = 6400
TILE_N = 2000


def _xsd_body(x_ref, wb_ref, wc_ref, xs_ref, xd_ref):
    xv = x_ref[...]
    xs_ref[...] = jnp.dot(xv, wb_ref[...], preferred_element_type=jnp.float32)
    xd_ref[...] = jnp.dot(xv, wc_ref[...], preferred_element_type=jnp.float32)


def _edge_body(ea_ref, g_ref, w1_ref, b1_ref, w2_ref, b2_ref, out_ref):
    w1 = w1_ref[...].astype(jnp.bfloat16)
    w2 = w2_ref[...].astype(jnp.bfloat16)
    t = (jnp.dot(ea_ref[...].astype(jnp.bfloat16), w1,
                 preferred_element_type=jnp.float32)
         + g_ref[...] + b1_ref[...])
    h = t * jax.nn.sigmoid(t)
    out_ref[...] = (jnp.dot(h.astype(jnp.bfloat16), w2,
                            preferred_element_type=jnp.float32)
                    + b2_ref[...])


def _node_body(x_ref, agg_ref, aggb_ref, w1a_ref, w1b_ref, b1_ref, w2_ref,
               b2_ref, out_ref):
    agg = (agg_ref[0] + agg_ref[1]) + (aggb_ref[0] + aggb_ref[1])
    t = (jnp.dot(x_ref[...], w1a_ref[...], preferred_element_type=jnp.float32)
         + jnp.dot(agg, w1b_ref[...], preferred_element_type=jnp.float32)
         + b1_ref[...])
    h = t * jax.nn.sigmoid(t)
    out_ref[...] = (jnp.dot(h, w2_ref[...], preferred_element_type=jnp.float32)
                    + b2_ref[...])


def _node_next_body(x_ref, agg_ref, aggb_ref, w1a_ref, w1b_ref, b1_ref,
                    w2_ref, b2_ref, wbn_ref, wcn_ref,
                    out_ref, xs_ref, xd_ref):
    agg = (agg_ref[0] + agg_ref[1]) + (aggb_ref[0] + aggb_ref[1])
    t = (jnp.dot(x_ref[...], w1a_ref[...], preferred_element_type=jnp.float32)
         + jnp.dot(agg, w1b_ref[...], preferred_element_type=jnp.float32)
         + b1_ref[...])
    h = t * jax.nn.sigmoid(t)
    xn = (jnp.dot(h, w2_ref[...], preferred_element_type=jnp.float32)
          + b2_ref[...])
    out_ref[...] = xn
    xs_ref[...] = jnp.dot(xn, wbn_ref[...],
                          preferred_element_type=jnp.float32)
    xd_ref[...] = jnp.dot(xn, wcn_ref[...],
                          preferred_element_type=jnp.float32)


def _run_xsd(x2, wb, wc):
    n, h = x2.shape
    wspec = pl.BlockSpec((h, h), lambda i: (0, 0))
    return pl.pallas_call(
        _xsd_body,
        grid=(n // TILE_N,),
        in_specs=[pl.BlockSpec((TILE_N, h), lambda i: (i, 0)), wspec, wspec],
        out_specs=[pl.BlockSpec((TILE_N, h), lambda i: (i, 0))] * 2,
        out_shape=[jax.ShapeDtypeStruct((n, h), jnp.float32)] * 2,
    )(x2, wb, wc)


def _run_edge_half(ea_full, ea_off, g, w1a, b1, w2, b2):
    eh, h = g.shape
    espec = pl.BlockSpec((TILE_E, h), lambda i: (i + ea_off, 0))
    gspec = pl.BlockSpec((TILE_E, h), lambda i: (i, 0))
    wspec = pl.BlockSpec((h, h), lambda i: (0, 0))
    bspec = pl.BlockSpec((1, h), lambda i: (0, 0))
    return pl.pallas_call(
        _edge_body,
        grid=(eh // TILE_E,),
        in_specs=[espec, gspec, wspec, bspec, wspec, bspec],
        out_specs=gspec,
        out_shape=jax.ShapeDtypeStruct((eh, h), jnp.float32),
    )(ea_full, g, w1a, b1, w2, b2)


def _run_edge_into(buf, ea_in, g, w1a, b1, w2, b2, out_off, e_total, in_off):
    eh, h = g.shape
    espec = pl.BlockSpec((TILE_E, h), lambda i: (i + in_off, 0))
    gspec = pl.BlockSpec((TILE_E, h), lambda i: (i, 0))
    ospec = pl.BlockSpec((TILE_E, h), lambda i: (i + out_off, 0))
    wspec = pl.BlockSpec((h, h), lambda i: (0, 0))
    bspec = pl.BlockSpec((1, h), lambda i: (0, 0))
    ins = [ea_in, g, w1a, b1, w2, b2]
    specs = [espec, gspec, wspec, bspec, wspec, bspec]
    if buf is None:
        body = _edge_body
        aliases = {}
    else:
        def body(_, *refs):
            _edge_body(*refs)
        ins = [buf] + ins
        specs = [pl.BlockSpec(memory_space=pltpu.MemorySpace.HBM)] + specs
        aliases = {0: 0}
    return pl.pallas_call(
        body,
        grid=(eh // TILE_E,),
        in_specs=specs,
        out_specs=ospec,
        out_shape=jax.ShapeDtypeStruct((e_total, h), jnp.float32),
        input_output_aliases=aliases,
    )(*ins)


def _run_node(x2, agg2a, agg2b, w1a, w1b, b1, w2, b2):
    n, h = x2.shape
    nspec = pl.BlockSpec((TILE_N, h), lambda i: (i, 0))
    aspec = pl.BlockSpec((2, TILE_N, h), lambda i: (0, i, 0))
    wspec = pl.BlockSpec((h, h), lambda i: (0, 0))
    bspec = pl.BlockSpec((1, h), lambda i: (0, 0))
    return pl.pallas_call(
        _node_body,
        grid=(n // TILE_N,),
        in_specs=[nspec, aspec, aspec, wspec, wspec, bspec, wspec, bspec],
        out_specs=nspec,
        out_shape=jax.ShapeDtypeStruct((n, h), jnp.float32),
    )(x2, agg2a, agg2b, w1a, w1b, b1, w2, b2)


def _run_node_next(x2, agg2a, agg2b, w1a, w1b, b1, w2, b2, wbn, wcn):
    n, h = x2.shape
    nspec = pl.BlockSpec((TILE_N, h), lambda i: (i, 0))
    aspec = pl.BlockSpec((2, TILE_N, h), lambda i: (0, i, 0))
    wspec = pl.BlockSpec((h, h), lambda i: (0, 0))
    bspec = pl.BlockSpec((1, h), lambda i: (0, 0))
    return pl.pallas_call(
        _node_next_body,
        grid=(n // TILE_N,),
        in_specs=[nspec, aspec, aspec, wspec, wspec, bspec, wspec, bspec,
                  wspec, wspec],
        out_specs=[nspec] * 3,
        out_shape=[jax.ShapeDtypeStruct((n, h), jnp.float32)] * 3,
    )(x2, agg2a, agg2b, w1a, w1b, b1, w2, b2, wbn, wcn)


def _make_gather(n_pad, e, h, k0):
    epw = e // NW
    nch = epw // CH
    tail = epw - nch * CH
    npair = nch // 2
    npc = n_pad // NS
    mesh = plsc.VectorSubcoreMesh(core_axis_name="c", subcore_axis_name="s")

    @functools.partial(
        pl.kernel, mesh=mesh,
        out_type=jax.ShapeDtypeStruct((e, h), jnp.float32),
        scratch_types=[
            pltpu.VMEM((epw,), jnp.int32),
            pltpu.VMEM((epw,), jnp.int32),
            pltpu.VMEM((CH, h), jnp.float32),
            pltpu.VMEM((CH, h), jnp.float32),
            pltpu.VMEM((CH, h), jnp.float32),
            pltpu.VMEM((CH, h), jnp.float32),
            pltpu.VMEM((tail, h), jnp.float32),
            pltpu.VMEM((tail, h), jnp.float32),
            pltpu.SemaphoreType.DMA,
            pltpu.SemaphoreType.DMA,
            pltpu.SemaphoreType.DMA,
        ])
    def gather(xs_hbm, xd_hbm, src_hbm, dst_hbm, g_hbm,
               sidx, didx, rs0, rd0, rs1, rd1, ts, td,
               sem0, sem1, semt):
        c = lax.axis_index("c")
        s = lax.axis_index("s")
        wid = s * NC + c
        base = wid * epw
        pltpu.sync_copy(src_hbm.at[pl.ds(k0 + base, epw)], sidx)
        pltpu.sync_copy(dst_hbm.at[pl.ds(k0 + base, epw)], didx)

        def issue(chunk, rs, rd, sem):
            off = pl.multiple_of(chunk * CH, CH)
            pltpu.async_copy(xs_hbm.at[sidx.at[pl.ds(off, CH)]], rs, sem)
            pltpu.async_copy(xd_hbm.at[didx.at[pl.ds(off, CH)]], rd, sem)

        def consume(chunk, rs, rd, sem):
            off = pl.multiple_of(chunk * CH, CH)
            pltpu.make_async_copy(
                xs_hbm.at[sidx.at[pl.ds(off, CH)]], rs, sem).wait()
            pltpu.make_async_copy(
                xd_hbm.at[didx.at[pl.ds(off, CH)]], rd, sem).wait()

            def add_row(i, carry2):
                for k in range(h // LANES):
                    sl = pl.ds(k * LANES, LANES)
                    rs[i, sl] = rs[i, sl] + rd[i, sl]
                return carry2

            lax.fori_loop(0, CH, add_row, 0)
            pltpu.sync_copy(rs, g_hbm.at[pl.ds(base + off, CH)])

        toff = nch * CH
        pltpu.async_copy(xs_hbm.at[sidx.at[pl.ds(toff, tail)]], ts, semt)
        pltpu.async_copy(xd_hbm.at[didx.at[pl.ds(toff, tail)]], td, semt)
        issue(0, rs0, rd0, sem0)

        def pair(j, carry):
            issue(2 * j + 1, rs1, rd1, sem1)
            consume(2 * j, rs0, rd0, sem0)

            @pl.when(2 * j + 2 < nch)
            def _():
                issue(2 * j + 2, rs0, rd0, sem0)

            consume(2 * j + 1, rs1, rd1, sem1)
            return carry

        lax.fori_loop(0, npair, pair, 0)
        if nch % 2 == 1:
            consume(nch - 1, rs0, rd0, sem0)

        pltpu.make_async_copy(
            xs_hbm.at[sidx.at[pl.ds(toff, tail)]], ts, semt).wait()
        pltpu.make_async_copy(
            xd_hbm.at[didx.at[pl.ds(toff, tail)]], td, semt).wait()

        def add_tail(i, carry2):
            for k in range(h // LANES):
                sl = pl.ds(k * LANES, LANES)
                ts[i, sl] = ts[i, sl] + td[i, sl]
            return carry2

        lax.fori_loop(0, tail, add_tail, 0)
        pltpu.sync_copy(ts, g_hbm.at[pl.ds(base + toff, tail)])

    return gather


def _make_scatter(n_pad, e, h, k0, ne_off):
    epw = e // NW
    npc = n_pad // NS
    mesh = plsc.VectorSubcoreMesh(core_axis_name="c", subcore_axis_name="s")

    nch = epw // CH
    tail = epw - nch * CH
    npair = nch // 2

    @functools.partial(
        pl.kernel, mesh=mesh,
        out_type=jax.ShapeDtypeStruct((NC, n_pad, h), jnp.float32),
        scratch_types=[
            pltpu.VMEM((CH,), jnp.int32),
            pltpu.VMEM((CH,), jnp.int32),
            pltpu.VMEM((tail,), jnp.int32),
            pltpu.VMEM((CH, h), jnp.float32),
            pltpu.VMEM((CH, h), jnp.float32),
            pltpu.VMEM((tail, h), jnp.float32),
            pltpu.VMEM_SHARED((n_pad, h), jnp.float32),
            pltpu.SemaphoreType.DMA,
            pltpu.SemaphoreType.DMA,
            pltpu.SemaphoreType.DMA,
        ])
    def scatter(ne_hbm, dst_hbm, zeros_hbm, agg_hbm,
                idx0, idx1, idxt, rows0, rows1, rowst, acc,
                sem0, sem1, semt):
        c = lax.axis_index("c")
        s = lax.axis_index("s")
        wid = s * NC + c
        base = wid * epw

        def issue(chunk, idxs, rows, sem, nrow):
            b = pl.multiple_of(chunk * CH, 8)
            pltpu.async_copy(dst_hbm.at[pl.ds(k0 + base + b, nrow)], idxs, sem)
            pltpu.async_copy(ne_hbm.at[pl.ds(ne_off + base + b, nrow)],
                             rows, sem)

        def consume(chunk, idxs, rows, sem, nrow):
            b = pl.multiple_of(chunk * CH, 8)
            pltpu.make_async_copy(
                dst_hbm.at[pl.ds(k0 + base + b, nrow)], idxs, sem).wait()
            pltpu.make_async_copy(
                ne_hbm.at[pl.ds(ne_off + base + b, nrow)], rows, sem).wait()
            pltpu.sync_copy(rows, acc.at[idxs], add=True)

        issue(nch, idxt, rowst, semt, tail)
        issue(0, idx0, rows0, sem0, CH)

        pltpu.sync_copy(zeros_hbm.at[pl.ds(s * npc, npc)],
                        acc.at[pl.ds(s * npc, npc)])
        plsc.subcore_barrier()

        def pair(j, carry):
            issue(2 * j + 1, idx1, rows1, sem1, CH)
            consume(2 * j, idx0, rows0, sem0, CH)

            @pl.when(2 * j + 2 < nch)
            def _():
                issue(2 * j + 2, idx0, rows0, sem0, CH)

            consume(2 * j + 1, idx1, rows1, sem1, CH)
            return carry

        lax.fori_loop(0, npair, pair, 0)
        if nch % 2 == 1:
            consume(nch - 1, idx0, rows0, sem0, CH)
        consume(nch, idxt, rowst, semt, tail)
        plsc.subcore_barrier()
        pltpu.sync_copy(acc.at[pl.ds(s * npc, npc)],
                        agg_hbm.at[c, pl.ds(s * npc, npc)])

    return scatter


def kernel(x, edge_index, edge_attr, node_positions,
           edge_W1, edge_b1, edge_W2, edge_b2,
           node_W1, node_b1, node_W2, node_b2):
    del node_positions
    b, n, h = x.shape
    e = edge_index.shape[1]
    n_layers = edge_W1.shape[0]

    x2 = x[0]
    src = edge_index[0]
    dst = edge_index[1]
    n_pad = -(-n // (8 * NS)) * (8 * NS)
    zeros_n = jnp.zeros((n_pad, h), jnp.float32)

    K = 2
    eh = e // K
    nblk = eh // TILE_E
    gathers = [_make_gather(n_pad, eh, h, k * eh) for k in range(K)]
    scat_half = [_make_scatter(n_pad, eh, h, k * eh, 0) for k in range(K)]
    scat_full = [_make_scatter(n_pad, eh, h, k * eh, k * eh) for k in range(K)]
    ea_full = edge_attr[0]
    ea_halves = None

    xs = xd = None
    for i in range(n_layers):
        w1a = edge_W1[i, :h]
        b1 = edge_b1[i][None]
        w2 = edge_W2[i]
        b2 = edge_b2[i][None]
        if xs is None:
            xs, xd = _run_xsd(x2, edge_W1[i, h:2 * h], edge_W1[i, 2 * h:])
        gs = [gathers[k](xs, xd, src, dst) for k in range(K)]
        if i < n_layers - 1:
            if ea_halves is None:
                ea_halves = [_run_edge_half(ea_full, k * nblk, gs[k], w1a, b1,
                                            w2, b2) for k in range(K)]
            else:
                ea_halves = [_run_edge_half(ea_halves[k], 0, gs[k], w1a, b1,
                                            w2, b2) for k in range(K)]
            aggs = [scat_half[k](ea_halves[k], dst, zeros_n)
                    for k in range(K)]
        else:
            buf = None
            for k in range(K):
                if ea_halves is not None:
                    ea_in, in_off = ea_halves[k], 0
                else:
                    ea_in, in_off = ea_full, k * nblk
                buf = _run_edge_into(buf, ea_in, gs[k], w1a, b1, w2, b2,
                                     k * nblk, e, in_off)
            ea_full = buf
            aggs = [scat_full[k](ea_full, dst, zeros_n) for k in range(K)]
        if i < n_layers - 1:
            x2, xs, xd = _run_node_next(
                x2, aggs[0], aggs[1], node_W1[i, :h], node_W1[i, h:],
                node_b1[i][None], node_W2[i], node_b2[i][None],
                edge_W1[i + 1, h:2 * h], edge_W1[i + 1, 2 * h:])
        else:
            x2 = _run_node(x2, aggs[0], aggs[1], node_W1[i, :h],
                           node_W1[i, h:], node_b1[i][None], node_W2[i],
                           node_b2[i][None])

    return (x2[None], ea_full[None])
